# Initial kernel scaffold; baseline (speedup 1.0000x reference)
import functools

import jax
import jax.numpy as jnp
from jax import lax
from jax.experimental import pallas as pl
from jax.experimental.pallas import tpu as pltpu
from jax.experimental.pallas import tpu_sc as plsc

N = 10000
E = 320000
D = 128

NC = 2
NS = 16
NW = NC * NS
EPT = E // NW
K = 80
NCHUNK = EPT // K
NPAD = 10240
WPT = NPAD // NS
RPT = N // NS

_mesh = plsc.VectorSubcoreMesh(core_axis_name="c", subcore_axis_name="s")


def _zero_f32(ref, n16):
    z = jnp.zeros((16,), jnp.float32)

    def body(i, _):
        r = i // 8
        c = i - r * 8
        ref[r, pl.ds(c * 16, 16)] = z
        return 0

    lax.fori_loop(0, n16, body, 0)


@functools.partial(
    pl.kernel,
    out_type=jax.ShapeDtypeStruct((NW, WPT), jnp.float32),
    mesh=_mesh,
    scratch_types=dict(
        deg_sh=pltpu.VMEM_SHARED((NPAD,), jnp.float32),
        zbuf=pltpu.VMEM((WPT,), jnp.float32),
        ones_v=pltpu.VMEM((K,), jnp.float32),
        didx=pltpu.VMEM((K,), jnp.int32),
    ),
)
def _deg_kernel(dst_hbm, out_hbm, deg_sh, zbuf, ones_v, didx):
    cid = lax.axis_index("c")
    sid = lax.axis_index("s")
    wid = sid * NC + cid

    one = jnp.ones((16,), jnp.float32)
    zero = jnp.zeros((16,), jnp.float32)

    def fill(i, _):
        zbuf[pl.ds(i * 16, 16)] = zero
        ones_v[pl.ds((i % 5) * 16, 16)] = one
        return 0

    lax.fori_loop(0, WPT // 16, fill, 0)

    pltpu.sync_copy(zbuf, deg_sh.at[pl.ds(sid * WPT, WPT)])
    plsc.subcore_barrier()

    def step(i, _):
        base = wid * EPT + i * K
        pltpu.sync_copy(dst_hbm.at[pl.ds(base, K)], didx)
        pltpu.sync_copy(ones_v, deg_sh.at[didx], add=True)
        return 0

    lax.fori_loop(0, NCHUNK, step, 0)
    plsc.subcore_barrier()

    pltpu.sync_copy(deg_sh.at[pl.ds(sid * WPT, WPT)],
                    out_hbm.at[cid * NS + sid])


@functools.partial(
    pl.kernel,
    out_type=jax.ShapeDtypeStruct((NC, N, D), jnp.float32),
    mesh=_mesh,
    scratch_types=dict(
        acc_sh=pltpu.VMEM_SHARED((N, D), jnp.float32),
        zbuf=pltpu.VMEM((RPT // 5, D), jnp.float32),
        rows=pltpu.VMEM((K, D), jnp.float32),
        sidx=pltpu.VMEM((K,), jnp.int32),
        didx=pltpu.VMEM((K,), jnp.int32),
        sem=pltpu.SemaphoreType.DMA,
    ),
)
def _scatter_kernel(xs_hbm, src_hbm, dst_hbm, out_hbm,
                    acc_sh, zbuf, rows, sidx, didx, sem):
    cid = lax.axis_index("c")
    sid = lax.axis_index("s")
    wid = sid * NC + cid

    _zero_f32(zbuf, (RPT // 5) * D // 16)
    for k in range(5):
        pltpu.sync_copy(
            zbuf, acc_sh.at[pl.ds(sid * RPT + k * (RPT // 5), RPT // 5), :])
    plsc.subcore_barrier()

    def step(i, _):
        base = wid * EPT + i * K
        pltpu.sync_copy(src_hbm.at[pl.ds(base, K)], sidx)
        pltpu.sync_copy(dst_hbm.at[pl.ds(base, K)], didx)
        pltpu.async_copy(xs_hbm.at[sidx], rows, sem).wait()
        pltpu.sync_copy(rows, acc_sh.at[didx], add=True)
        return 0

    lax.fori_loop(0, NCHUNK, step, 0)
    plsc.subcore_barrier()

    pltpu.sync_copy(acc_sh.at[pl.ds(sid * RPT, RPT), :],
                    out_hbm.at[cid, pl.ds(sid * RPT, RPT), :])


RB = 400
GRID = N // RB


def _mm1_body(x_ref, w_ref, degA_ref, degB_ref, xs_ref, dis_ref):
    dis = lax.rsqrt(degA_ref[...] + degB_ref[...] + 1.0)
    xs_ref[...] = jnp.dot(x_ref[...], w_ref[...],
                          preferred_element_type=jnp.float32) * dis
    dis_ref[...] = dis


def _mm1(x, W1, degA, degB):
    return pl.pallas_call(
        _mm1_body,
        grid=(GRID,),
        in_specs=[
            pl.BlockSpec((RB, D), lambda i: (i, 0)),
            pl.BlockSpec((D, D), lambda i: (0, 0)),
            pl.BlockSpec((RB, 1), lambda i: (i, 0)),
            pl.BlockSpec((RB, 1), lambda i: (i, 0)),
        ],
        out_specs=[
            pl.BlockSpec((RB, D), lambda i: (i, 0)),
            pl.BlockSpec((RB, 1), lambda i: (i, 0)),
        ],
        out_shape=[
            jax.ShapeDtypeStruct((N, D), jnp.float32),
            jax.ShapeDtypeStruct((N, 1), jnp.float32),
        ],
    )(x, W1, degA, degB)


def _mm2_body(aA_ref, aB_ref, xs_ref, dis_ref, b_ref, w_ref, out_ref):
    dis = dis_ref[...]
    h = (aA_ref[...] + aB_ref[...] + xs_ref[...]) * dis + b_ref[...]
    h = jnp.maximum(h, 0.0)
    out_ref[...] = jnp.dot(h, w_ref[...],
                           preferred_element_type=jnp.float32) * dis


def _mm2(accA, accB, xs, dis, b, W2):
    return pl.pallas_call(
        _mm2_body,
        grid=(GRID,),
        in_specs=[
            pl.BlockSpec((RB, D), lambda i: (i, 0)),
            pl.BlockSpec((RB, D), lambda i: (i, 0)),
            pl.BlockSpec((RB, D), lambda i: (i, 0)),
            pl.BlockSpec((RB, 1), lambda i: (i, 0)),
            pl.BlockSpec((1, D), lambda i: (0, 0)),
            pl.BlockSpec((D, D), lambda i: (0, 0)),
        ],
        out_specs=pl.BlockSpec((RB, D), lambda i: (i, 0)),
        out_shape=jax.ShapeDtypeStruct((N, D), jnp.float32),
    )(accA, accB, xs, dis, b, W2)


def _fin_body(aA_ref, aB_ref, xs_ref, dis_ref, b_ref, out_ref):
    out_ref[...] = ((aA_ref[...] + aB_ref[...] + xs_ref[...]) * dis_ref[...]
                    + b_ref[...])


def _fin(accA, accB, xs, dis, b):
    return pl.pallas_call(
        _fin_body,
        grid=(GRID,),
        in_specs=[
            pl.BlockSpec((RB, D), lambda i: (i, 0)),
            pl.BlockSpec((RB, D), lambda i: (i, 0)),
            pl.BlockSpec((RB, D), lambda i: (i, 0)),
            pl.BlockSpec((RB, 1), lambda i: (i, 0)),
            pl.BlockSpec((1, D), lambda i: (0, 0)),
        ],
        out_specs=pl.BlockSpec((RB, D), lambda i: (i, 0)),
        out_shape=jax.ShapeDtypeStruct((N, D), jnp.float32),
    )(accA, accB, xs, dis, b)


def kernel(x, edge_index, W1, b1, W2, b2):
    src = edge_index[0]
    dst = edge_index[1]

    deg_p = _deg_kernel(dst)
    degA = deg_p[:NS].reshape(NPAD)[:N, None]
    degB = deg_p[NS:].reshape(NPAD)[:N, None]

    xs1, dis = _mm1(x, W1, degA, degB)
    acc1 = _scatter_kernel(xs1, src, dst)
    xs2 = _mm2(acc1[0], acc1[1], xs1, dis, b1[None, :], W2)
    acc2 = _scatter_kernel(xs2, src, dst)
    return _fin(acc2[0], acc2[1], xs2, dis, b2[None, :])


# speedup vs baseline: 12.4292x; 12.4292x over previous
import functools

import jax
import jax.numpy as jnp
from jax import lax
from jax.experimental import pallas as pl
from jax.experimental.pallas import tpu as pltpu
from jax.experimental.pallas import tpu_sc as plsc

N = 10000
E = 320000
D = 128

NC = 2
NS = 16
NW = NC * NS
EPT = E // NW
K = 80
NCHUNK = EPT // K
NPAD = 10240
WPT = NPAD // NS
RPT = NPAD // NS

_mesh = plsc.VectorSubcoreMesh(
    core_axis_name="c", subcore_axis_name="s", num_cores=NC, num_subcores=NS)


def _zero_f32(ref, n16):
    z = jnp.zeros((16,), jnp.float32)

    def body(i, _):
        r = i // 8
        c = i - r * 8
        ref[r, pl.ds(c * 16, 16)] = z
        return 0

    lax.fori_loop(0, n16, body, 0)


@functools.partial(
    pl.kernel,
    out_type=jax.ShapeDtypeStruct((NW, WPT), jnp.float32),
    mesh=_mesh,
    scratch_types=dict(
        deg_sh=pltpu.VMEM_SHARED((NPAD,), jnp.float32),
        zbuf=pltpu.VMEM((WPT,), jnp.float32),
        ones_v=pltpu.VMEM((K,), jnp.float32),
        didx=pltpu.VMEM((K,), jnp.int32),
    ),
)
def _deg_kernel(dst_hbm, out_hbm, deg_sh, zbuf, ones_v, didx):
    cid = lax.axis_index("c")
    sid = lax.axis_index("s")
    wid = sid * NC + cid

    one = jnp.ones((16,), jnp.float32)
    zero = jnp.zeros((16,), jnp.float32)

    def fill(i, _):
        zbuf[pl.ds(i * 16, 16)] = zero
        ones_v[pl.ds((i % 5) * 16, 16)] = one
        return 0

    lax.fori_loop(0, WPT // 16, fill, 0)

    pltpu.sync_copy(zbuf, deg_sh.at[pl.ds(sid * WPT, WPT)])
    plsc.subcore_barrier()

    def step(i, _):
        base = wid * EPT + i * K
        pltpu.sync_copy(dst_hbm.at[pl.ds(base, K)], didx)
        pltpu.sync_copy(ones_v, deg_sh.at[didx], add=True)
        return 0

    lax.fori_loop(0, NCHUNK, step, 0)
    plsc.subcore_barrier()

    pltpu.sync_copy(deg_sh.at[pl.ds(sid * WPT, WPT)],
                    out_hbm.at[cid * NS + sid])


@functools.partial(
    pl.kernel,
    out_type=jax.ShapeDtypeStruct((NC, NPAD, D), jnp.float32),
    mesh=_mesh,
    scratch_types=dict(
        acc_sh=pltpu.VMEM_SHARED((NPAD, D), jnp.float32),
        zbuf=pltpu.VMEM((RPT // 5, D), jnp.float32),
        rows=pltpu.VMEM((K, D), jnp.float32),
        sidx=pltpu.VMEM((K,), jnp.int32),
        didx=pltpu.VMEM((K,), jnp.int32),
        sem=pltpu.SemaphoreType.DMA,
    ),
)
def _scatter_kernel(xs_hbm, src_hbm, dst_hbm, out_hbm,
                    acc_sh, zbuf, rows, sidx, didx, sem):
    cid = lax.axis_index("c")
    sid = lax.axis_index("s")
    wid = sid * NC + cid

    _zero_f32(zbuf, (RPT // 5) * D // 16)
    for k in range(5):
        pltpu.sync_copy(
            zbuf, acc_sh.at[pl.ds(sid * RPT + k * (RPT // 5), RPT // 5), :])
    plsc.subcore_barrier()

    def step(i, _):
        base = wid * EPT + i * K
        pltpu.sync_copy(src_hbm.at[pl.ds(base, K)], sidx)
        pltpu.sync_copy(dst_hbm.at[pl.ds(base, K)], didx)
        pltpu.async_copy(xs_hbm.at[sidx], rows, sem).wait()
        pltpu.sync_copy(rows, acc_sh.at[didx], add=True)
        return 0

    lax.fori_loop(0, NCHUNK, step, 0)
    plsc.subcore_barrier()

    pltpu.sync_copy(acc_sh.at[pl.ds(sid * RPT, RPT), :],
                    out_hbm.at[cid, pl.ds(sid * RPT, RPT), :])


RB = 400
GRID = N // RB


def _mm1_body(x_ref, w_ref, degA_ref, degB_ref, xs_ref, dis_ref):
    dis = lax.rsqrt(degA_ref[...] + degB_ref[...] + 1.0)
    xs_ref[...] = jnp.dot(x_ref[...], w_ref[...],
                          preferred_element_type=jnp.float32) * dis
    dis_ref[...] = dis


def _mm1(x, W1, degA, degB):
    return pl.pallas_call(
        _mm1_body,
        grid=(GRID,),
        in_specs=[
            pl.BlockSpec((RB, D), lambda i: (i, 0)),
            pl.BlockSpec((D, D), lambda i: (0, 0)),
            pl.BlockSpec((RB, 1), lambda i: (i, 0)),
            pl.BlockSpec((RB, 1), lambda i: (i, 0)),
        ],
        out_specs=[
            pl.BlockSpec((RB, D), lambda i: (i, 0)),
            pl.BlockSpec((RB, 1), lambda i: (i, 0)),
        ],
        out_shape=[
            jax.ShapeDtypeStruct((N, D), jnp.float32),
            jax.ShapeDtypeStruct((N, 1), jnp.float32),
        ],
    )(x, W1, degA, degB)


def _mm2_body(aA_ref, aB_ref, xs_ref, dis_ref, b_ref, w_ref, out_ref):
    dis = dis_ref[...]
    h = (aA_ref[...] + aB_ref[...] + xs_ref[...]) * dis + b_ref[...]
    h = jnp.maximum(h, 0.0)
    out_ref[...] = jnp.dot(h, w_ref[...],
                           preferred_element_type=jnp.float32) * dis


def _mm2(accA, accB, xs, dis, b, W2):
    return pl.pallas_call(
        _mm2_body,
        grid=(GRID,),
        in_specs=[
            pl.BlockSpec((RB, D), lambda i: (i, 0)),
            pl.BlockSpec((RB, D), lambda i: (i, 0)),
            pl.BlockSpec((RB, D), lambda i: (i, 0)),
            pl.BlockSpec((RB, 1), lambda i: (i, 0)),
            pl.BlockSpec((1, D), lambda i: (0, 0)),
            pl.BlockSpec((D, D), lambda i: (0, 0)),
        ],
        out_specs=pl.BlockSpec((RB, D), lambda i: (i, 0)),
        out_shape=jax.ShapeDtypeStruct((N, D), jnp.float32),
    )(accA, accB, xs, dis, b, W2)


def _fin_body(aA_ref, aB_ref, xs_ref, dis_ref, b_ref, out_ref):
    out_ref[...] = ((aA_ref[...] + aB_ref[...] + xs_ref[...]) * dis_ref[...]
                    + b_ref[...])


def _fin(accA, accB, xs, dis, b):
    return pl.pallas_call(
        _fin_body,
        grid=(GRID,),
        in_specs=[
            pl.BlockSpec((RB, D), lambda i: (i, 0)),
            pl.BlockSpec((RB, D), lambda i: (i, 0)),
            pl.BlockSpec((RB, D), lambda i: (i, 0)),
            pl.BlockSpec((RB, 1), lambda i: (i, 0)),
            pl.BlockSpec((1, D), lambda i: (0, 0)),
        ],
        out_specs=pl.BlockSpec((RB, D), lambda i: (i, 0)),
        out_shape=jax.ShapeDtypeStruct((N, D), jnp.float32),
    )(accA, accB, xs, dis, b)


def kernel(x, edge_index, W1, b1, W2, b2):
    src = edge_index[0]
    dst = edge_index[1]

    deg_p = _deg_kernel(dst)
    degA = deg_p[:NS].reshape(NPAD)[:N, None]
    degB = deg_p[NS:].reshape(NPAD)[:N, None]

    xs1, dis = _mm1(x, W1, degA, degB)
    acc1 = _scatter_kernel(xs1, src, dst)
    xs2 = _mm2(acc1[0, :N], acc1[1, :N], xs1, dis, b1[None, :], W2)
    acc2 = _scatter_kernel(xs2, src, dst)
    return _fin(acc2[0, :N], acc2[1, :N], xs2, dis, b2[None, :])
